# Initial kernel scaffold; baseline (speedup 1.0000x reference)
import jax
import jax.numpy as jnp
from jax.experimental import pallas as pl


def kernel(x, table):
    raise NotImplementedError("write your pallas kernel here")


# speedup vs baseline: 1.5217x; 1.5217x over previous
import functools

import jax
import jax.numpy as jnp
from jax import lax
from jax.experimental import pallas as pl
from jax.experimental.pallas import tpu as pltpu
from jax.experimental.pallas import tpu_sc as plsc

DIM = 32
ROWS = 16384
COLS = 26
NC = 2
NS = 16
NW = NC * NS
B = ROWS * COLS
PER_W = B // NW
CH = 128
NCH = PER_W // CH
NBUF = 2


def _mesh():
    return plsc.VectorSubcoreMesh(core_axis_name="c", subcore_axis_name="s")


@functools.partial(
    pl.kernel,
    out_type=jax.ShapeDtypeStruct((B, DIM), jnp.float32),
    mesh=plsc.VectorSubcoreMesh(core_axis_name="c", subcore_axis_name="s"),
    scratch_types=[
        pltpu.VMEM((NCH, CH), jnp.int32),
        pltpu.VMEM((CH, DIM), jnp.float32),
        pltpu.VMEM((CH, DIM), jnp.float32),
        pltpu.SemaphoreType.DMA,
        pltpu.SemaphoreType.DMA,
    ],
    compiler_params=pltpu.CompilerParams(use_tc_tiling_on_sc=False),
)
def _sc_gather(idx_hbm, table_hbm, out_hbm, idx_v, rows0, rows1, sem0, sem1):
    wid = lax.axis_index("s") * NC + lax.axis_index("c")
    base = wid * PER_W

    pltpu.sync_copy(idx_hbm.at[wid], idx_v)

    rows = (rows0, rows1)
    sems = (sem0, sem1)

    for b in range(NBUF):
        pltpu.async_copy(table_hbm.at[idx_v.at[b]], rows[b], sems[b])

    def step(j, b):
        pltpu.make_async_copy(table_hbm.at[idx_v.at[j]], rows[b], sems[b]).wait()
        pltpu.sync_copy(rows[b], out_hbm.at[pl.ds(base + j * CH, CH)])

    def body(i, carry):
        for b in range(NBUF):
            j = i * NBUF + b
            step(j, b)
            pltpu.async_copy(
                table_hbm.at[idx_v.at[j + NBUF]], rows[b], sems[b])
        return carry

    lax.fori_loop(0, (NCH - NBUF) // NBUF, body, 0)

    for b in range(NBUF):
        step(NCH - NBUF + b, b)


def kernel(x, table):
    idx = x.reshape(NW, NCH, CH)
    out = _sc_gather(idx, table)
    return out.reshape(ROWS, COLS, DIM)


# speedup vs baseline: 1.5752x; 1.0352x over previous
import functools

import jax
import jax.numpy as jnp
from jax import lax
from jax.experimental import pallas as pl
from jax.experimental.pallas import tpu as pltpu
from jax.experimental.pallas import tpu_sc as plsc

DIM = 32
ROWS = 16384
COLS = 26
NC = 2
NS = 16
NW = NC * NS
B = ROWS * COLS
PER_W = B // NW
CH = 1664
NCH = PER_W // CH
NBUF = 2


def _mesh():
    return plsc.VectorSubcoreMesh(core_axis_name="c", subcore_axis_name="s")


@functools.partial(
    pl.kernel,
    out_type=jax.ShapeDtypeStruct((B, DIM), jnp.float32),
    mesh=plsc.VectorSubcoreMesh(core_axis_name="c", subcore_axis_name="s"),
    scratch_types=[
        pltpu.VMEM((NCH, CH), jnp.int32),
        pltpu.VMEM((CH, DIM), jnp.float32),
        pltpu.VMEM((CH, DIM), jnp.float32),
        pltpu.SemaphoreType.DMA,
        pltpu.SemaphoreType.DMA,
    ],
    compiler_params=pltpu.CompilerParams(use_tc_tiling_on_sc=False),
)
def _sc_gather(idx_hbm, table_hbm, out_hbm, idx_v, rows0, rows1, sem0, sem1):
    wid = lax.axis_index("s") * NC + lax.axis_index("c")
    base = wid * PER_W

    pltpu.sync_copy(idx_hbm.at[wid], idx_v)

    rows = (rows0, rows1)
    sems = (sem0, sem1)

    for b in range(NBUF):
        pltpu.async_copy(table_hbm.at[idx_v.at[b]], rows[b], sems[b])

    def step(j, b):
        pltpu.make_async_copy(table_hbm.at[idx_v.at[j]], rows[b], sems[b]).wait()
        pltpu.sync_copy(rows[b], out_hbm.at[pl.ds(base + j * CH, CH)])

    def body(i, carry):
        for b in range(NBUF):
            j = i * NBUF + b
            step(j, b)
            pltpu.async_copy(
                table_hbm.at[idx_v.at[j + NBUF]], rows[b], sems[b])
        return carry

    lax.fori_loop(0, (NCH - NBUF) // NBUF, body, 0)

    for b in range(NBUF):
        step(NCH - NBUF + b, b)


def kernel(x, table):
    idx = x.reshape(NW, NCH, CH)
    out = _sc_gather(idx, table)
    return out.reshape(ROWS, COLS, DIM)


# speedup vs baseline: 1.5776x; 1.0015x over previous
import functools

import jax
import jax.numpy as jnp
from jax import lax
from jax.experimental import pallas as pl
from jax.experimental.pallas import tpu as pltpu
from jax.experimental.pallas import tpu_sc as plsc

DIM = 32
ROWS = 16384
COLS = 26
NC = 2
NS = 16
NW = NC * NS
B = ROWS * COLS
PER_W = B // NW
CH = 832
NCH = PER_W // CH
NBUF = 4


def _mesh():
    return plsc.VectorSubcoreMesh(core_axis_name="c", subcore_axis_name="s")


@functools.partial(
    pl.kernel,
    out_type=jax.ShapeDtypeStruct((B, DIM), jnp.float32),
    mesh=plsc.VectorSubcoreMesh(core_axis_name="c", subcore_axis_name="s"),
    scratch_types=(
        [pltpu.VMEM((NCH, CH), jnp.int32)]
        + [pltpu.VMEM((CH, DIM), jnp.float32) for _ in range(NBUF)]
        + [pltpu.SemaphoreType.DMA for _ in range(NBUF)]
    ),
    compiler_params=pltpu.CompilerParams(use_tc_tiling_on_sc=False),
)
def _sc_gather(idx_hbm, table_hbm, out_hbm, idx_v, *bufs):
    wid = lax.axis_index("s") * NC + lax.axis_index("c")
    base = wid * PER_W

    pltpu.sync_copy(idx_hbm.at[wid], idx_v)

    rows = bufs[:NBUF]
    sems = bufs[NBUF:]

    for b in range(NBUF):
        pltpu.async_copy(table_hbm.at[idx_v.at[b]], rows[b], sems[b])

    def step(j, b):
        pltpu.make_async_copy(table_hbm.at[idx_v.at[j]], rows[b], sems[b]).wait()
        pltpu.sync_copy(rows[b], out_hbm.at[pl.ds(base + j * CH, CH)])

    def body(i, carry):
        for b in range(NBUF):
            j = i * NBUF + b
            step(j, b)
            pltpu.async_copy(
                table_hbm.at[idx_v.at[j + NBUF]], rows[b], sems[b])
        return carry

    lax.fori_loop(0, (NCH - NBUF) // NBUF, body, 0)

    for b in range(NBUF):
        step(NCH - NBUF + b, b)


def kernel(x, table):
    idx = x.reshape(NW, NCH, CH)
    out = _sc_gather(idx, table)
    return out.reshape(ROWS, COLS, DIM)


# speedup vs baseline: 1.5815x; 1.0025x over previous
import functools

import jax
import jax.numpy as jnp
from jax import lax
from jax.experimental import pallas as pl
from jax.experimental.pallas import tpu as pltpu
from jax.experimental.pallas import tpu_sc as plsc

VOC = 1_000_000
DIM = 32
ROWS = 16384
COLS = 26
NC = 2
NS = 16
NW = NC * NS
BPW = ROWS // NW
NBUF = 4


@functools.partial(
    pl.kernel,
    out_type=jax.ShapeDtypeStruct((ROWS, COLS, DIM), jnp.float32),
    mesh=plsc.VectorSubcoreMesh(core_axis_name="c", subcore_axis_name="s"),
    scratch_types=(
        [pltpu.VMEM((COLS, BPW), jnp.int32)]
        + [pltpu.VMEM((BPW, DIM), jnp.float32) for _ in range(NBUF)]
        + [pltpu.SemaphoreType.DMA for _ in range(NBUF)]
    ),
    compiler_params=pltpu.CompilerParams(use_tc_tiling_on_sc=False),
)
def _sc_gather(xt_hbm, table_hbm, out_hbm, idx_v, *bufs):
    wid = lax.axis_index("s") * NC + lax.axis_index("c")
    b0 = wid * BPW

    pltpu.sync_copy(xt_hbm.at[:, pl.ds(b0, BPW)], idx_v)

    rows = bufs[:NBUF]
    sems = bufs[NBUF:]

    for b in range(NBUF):
        pltpu.async_copy(table_hbm.at[idx_v.at[b]], rows[b], sems[b])

    def step(j, b):
        pltpu.make_async_copy(
            table_hbm.at[idx_v.at[j]], rows[b], sems[b]).wait()
        pltpu.sync_copy(rows[b], out_hbm.at[pl.ds(b0, BPW), j])

    def body(i, carry):
        for b in range(NBUF):
            j = i * NBUF + b
            step(j, b)
            pltpu.async_copy(
                table_hbm.at[idx_v.at[j + NBUF]], rows[b], sems[b])
        return carry

    steady = (COLS - NBUF) // NBUF
    lax.fori_loop(0, steady, body, 0)
    for k in range(steady * NBUF, COLS - NBUF):
        b = k % NBUF
        step(k, b)
        pltpu.async_copy(
            table_hbm.at[idx_v.at[k + NBUF]], rows[b], sems[b])
    for j in range(COLS - NBUF, COLS):
        step(j, j % NBUF)


def kernel(x, table):
    return _sc_gather(x.T, table)
